# baseline (speedup 1.0000x reference)
import functools

import jax
import jax.numpy as jnp
from jax import lax
from jax.experimental import pallas as pl
from jax.experimental.pallas import tpu as pltpu
from jax.experimental.pallas import tpu_sc as plsc

N = 10000
E = 320000
HID = 32
PACK = 4
R4 = N // PACK
PW = PACK * HID
NC = 2
NS = 16
NW = NC * NS
EPT = E // NW
CHUNK = 1000
NBLK = EPT // CHUNK
WCH = 1000
NWCH = N // WCH

_MESH = plsc.VectorSubcoreMesh(core_axis_name="c", subcore_axis_name="s")


@functools.partial(
    pl.kernel,
    out_type=jax.ShapeDtypeStruct((NC * N, HID), jnp.float32),
    mesh=_MESH,
    scratch_types=[
        pltpu.VMEM_SHARED((N, HID), jnp.float32),
        pltpu.VMEM((NBLK, CHUNK), jnp.int32),
        pltpu.VMEM((CHUNK, HID), jnp.float32),
        pltpu.SemaphoreType.DMA,
    ],
    compiler_params=pltpu.CompilerParams(use_tc_tiling_on_sc=False),
)
def _degree_hist(dst_hbm, ones_hbm, zrows_hbm, out_hbm, acc, dsti, onesv,
                 semi):
    c = lax.axis_index("c")
    s = lax.axis_index("s")
    wid = c * NS + s
    cpd = pltpu.async_copy(dst_hbm.at[wid], dsti, semi)
    cpo = pltpu.async_copy(ones_hbm, onesv, semi)

    @pl.when(s < NWCH)
    def _zero():
        pltpu.sync_copy(zrows_hbm, acc.at[pl.ds(s * WCH, WCH)])

    cpd.wait()
    cpo.wait()
    plsc.subcore_barrier()
    scats = []
    for blk in range(NBLK):
        scats.append(pltpu.async_copy(onesv, acc.at[dsti.at[blk]], semi,
                                      add=True))
    for cp in scats:
        cp.wait()
    plsc.subcore_barrier()

    @pl.when(s < NWCH)
    def _writeout():
        pltpu.sync_copy(acc.at[pl.ds(s * WCH, WCH)],
                        out_hbm.at[pl.ds(c * N + s * WCH, WCH)])


@functools.partial(
    pl.kernel,
    out_type=jax.ShapeDtypeStruct((NC * N, HID), jnp.float32),
    mesh=_MESH,
    scratch_types=[
        pltpu.VMEM_SHARED((N, HID), jnp.float32),
        pltpu.VMEM_SHARED((N, HID), jnp.float32),
        pltpu.VMEM((NBLK, CHUNK), jnp.int32),
        pltpu.VMEM((NBLK, CHUNK), jnp.int32),
        pltpu.VMEM((2, CHUNK, HID), jnp.float32),
        pltpu.SemaphoreType.DMA,
        pltpu.SemaphoreType.DMA,
        pltpu.SemaphoreType.DMA,
        pltpu.SemaphoreType.DMA,
        pltpu.SemaphoreType.DMA,
    ],
    compiler_params=pltpu.CompilerParams(use_tc_tiling_on_sc=False),
)
def _edge_aggregate(table_hbm, src_hbm, dst_hbm, zrows_hbm, out_hbm,
                    acc, tbl, srci, dsti, rowsv, semi, semg0, semg1,
                    sems0, sems1):
    c = lax.axis_index("c")
    s = lax.axis_index("s")
    wid = c * NS + s
    semg = (semg0, semg1)
    sems = (sems0, sems1)
    NBUF = 2

    cpi = pltpu.async_copy(src_hbm.at[wid], srci, semi)
    cpd = pltpu.async_copy(dst_hbm.at[wid], dsti, semi)

    @pl.when(s < NWCH)
    def _stage():
        pltpu.sync_copy(zrows_hbm, acc.at[pl.ds(s * WCH, WCH)])
        pltpu.sync_copy(table_hbm.at[pl.ds(s * WCH, WCH)],
                        tbl.at[pl.ds(s * WCH, WCH)])

    cpi.wait()
    cpd.wait()
    plsc.subcore_barrier()
    gathers = [None] * NBUF
    scats = [None] * NBUF
    gathers[0] = pltpu.async_copy(tbl.at[srci.at[0]], rowsv.at[0],
                                  semg[0])
    for blk in range(NBLK):
        p = blk % NBUF
        gathers[p].wait()
        scats[p] = pltpu.async_copy(rowsv.at[p], acc.at[dsti.at[blk]],
                                    sems[p], add=True)
        if blk + 1 < NBLK:
            q = (blk + 1) % NBUF
            if blk >= 1:
                scats[q].wait()
            gathers[q] = pltpu.async_copy(tbl.at[srci.at[blk + 1]],
                                          rowsv.at[q], semg[q])
    for blk in range(NBLK - NBUF, NBLK):
        scats[blk % NBUF].wait()
    plsc.subcore_barrier()

    @pl.when(s < NWCH)
    def _writeout():
        pltpu.sync_copy(acc.at[pl.ds(s * WCH, WCH)],
                        out_hbm.at[pl.ds(c * N + s * WCH, WCH)])


def _xw1_body(x4_ref, w14_ref, xw_ref):
    xw_ref[...] = jnp.dot(x4_ref[...], w14_ref[...],
                          preferred_element_type=jnp.float32)


def _xw1(x4, w14):
    return pl.pallas_call(
        _xw1_body,
        out_shape=jax.ShapeDtypeStruct((R4, PW), jnp.float32),
    )(x4, w14)


def _prep_body(degp_ref, xw_ref, dinv_ref, xws_ref):
    deg = degp_ref[0] + degp_ref[1] + 1.0
    dinv = lax.rsqrt(deg)
    dinv_ref[...] = dinv
    xws_ref[...] = xw_ref[...] * dinv


def _prep(degp, xw):
    return pl.pallas_call(
        _prep_body,
        out_shape=[
            jax.ShapeDtypeStruct((R4, PW), jnp.float32),
            jax.ShapeDtypeStruct((R4, PW), jnp.float32),
        ],
    )(degp, xw)


def _boundary_body(has_jk, p_ref, xws_ref, dinv_ref, b_ref, w_ref, jk_ref,
                   xwsn_ref, jko_ref):
    total = p_ref[0] + p_ref[1] + xws_ref[...]
    h = jnp.maximum(total * dinv_ref[...] + b_ref[...], 0.0)
    jko = jnp.maximum(jk_ref[...], h) if has_jk else h
    jko_ref[...] = jko
    xwsn_ref[...] = jnp.dot(h, w_ref[...],
                            preferred_element_type=jnp.float32) * dinv_ref[...]


def _boundary(partials, xws, dinv, b, w_next, jk):
    has_jk = jk is not None
    args = [partials, xws, dinv, b, w_next] + ([jk] if has_jk else [])
    if has_jk:
        body = functools.partial(_boundary_body, True)
    else:
        def body(p, xw, di, bb, ww, xn, jo):
            _boundary_body(False, p, xw, di, bb, ww, None, xn, jo)
    return pl.pallas_call(
        body,
        out_shape=[
            jax.ShapeDtypeStruct((R4, PW), jnp.float32),
            jax.ShapeDtypeStruct((R4, PW), jnp.float32),
        ],
    )(*args)


def _final_body(p_ref, xws_ref, dinv_ref, b_ref, jk_ref, fcw_ref,
                fcb_ref, bones_ref, out_ref):
    total = p_ref[0] + p_ref[1] + xws_ref[...]
    h = jnp.maximum(total * dinv_ref[...] + b_ref[...], 0.0)
    jk = jnp.maximum(jk_ref[...], h)
    logits = jnp.dot(jk, fcw_ref[...],
                     preferred_element_type=jnp.float32) + fcb_ref[...]
    m = jnp.max(logits, axis=1, keepdims=True)
    z = logits - m
    ssum = jnp.dot(jnp.exp(z), bones_ref[...],
                   preferred_element_type=jnp.float32)
    out_ref[...] = z - jnp.log(ssum)


def _final(partials, xws, dinv, b, jk, fcw4, fcb4, bones):
    nc4 = fcw4.shape[1]
    return pl.pallas_call(
        _final_body,
        out_shape=jax.ShapeDtypeStruct((R4, nc4), jnp.float32),
    )(partials, xws, dinv, b, jk, fcw4, fcb4, bones)


def _blkdiag(w):
    a, b = w.shape
    out = jnp.zeros((PACK * a, PACK * b), w.dtype)
    for j in range(PACK):
        out = out.at[j * a:(j + 1) * a, j * b:(j + 1) * b].set(w)
    return out


def kernel(x, edge_index, Ws, bs, fcW, fcb):
    src3 = edge_index[0].reshape(NW, NBLK, CHUNK)
    dst3 = edge_index[1].reshape(NW, NBLK, CHUNK)
    ones_r = jnp.ones((CHUNK, HID), jnp.float32)
    zeros_r = jnp.zeros((WCH, HID), jnp.float32)

    w14 = _blkdiag(Ws[0])
    w4s = [_blkdiag(w) for w in Ws[1:]]
    fcw4 = _blkdiag(fcW)
    b4s = [jnp.tile(b, PACK).reshape(1, PW) for b in bs]
    fcb4 = jnp.tile(fcb, PACK).reshape(1, -1)

    nclass = fcW.shape[1]
    bones = _blkdiag(jnp.ones((nclass, nclass), jnp.float32))

    deg4 = _degree_hist(dst3, ones_r, zeros_r).reshape(NC, R4, PW)
    x4 = x.reshape(R4, PACK * x.shape[1])
    xw1 = _xw1(x4, w14)
    dinv, xws = _prep(deg4, xw1)

    jk = None
    nlayers = len(Ws)
    for l in range(nlayers):
        partials = _edge_aggregate(xws.reshape(N, HID), src3, dst3, zeros_r)
        p4 = partials.reshape(NC, R4, PW)
        if l + 1 < nlayers:
            xws, jk = _boundary(p4, xws, dinv, b4s[l], w4s[l], jk)
        else:
            out4 = _final(p4, xws, dinv, b4s[l], jk, fcw4, fcb4, bones)
    return out4.reshape(N, nclass)


# speedup vs baseline: 1.0934x; 1.0934x over previous
import functools

import jax
import jax.numpy as jnp
from jax import lax
from jax.experimental import pallas as pl
from jax.experimental.pallas import tpu as pltpu
from jax.experimental.pallas import tpu_sc as plsc

N = 10000
E = 320000
HID = 32
PACK = 4
R4 = N // PACK
PW = PACK * HID
NC = 2
NS = 16
NW = NC * NS
EPT = E // NW
CHUNK = 1000
NBLK = EPT // CHUNK
WCH = 1000
NWCH = N // WCH

_MESH = plsc.VectorSubcoreMesh(core_axis_name="c", subcore_axis_name="s")


@functools.partial(
    pl.kernel,
    out_type=jax.ShapeDtypeStruct((NC * N, HID), jnp.float32),
    mesh=_MESH,
    scratch_types=[
        pltpu.VMEM_SHARED((N, HID), jnp.float32),
        pltpu.VMEM((NBLK, CHUNK), jnp.int32),
        pltpu.VMEM((CHUNK, HID), jnp.float32),
        pltpu.SemaphoreType.DMA,
    ],
    compiler_params=pltpu.CompilerParams(use_tc_tiling_on_sc=False),
)
def _degree_hist(ei_hbm, ones_hbm, zrows_hbm, out_hbm, acc, dsti, onesv,
                 semi):
    c = lax.axis_index("c")
    s = lax.axis_index("s")
    wid = c * NS + s
    cpd = pltpu.async_copy(ei_hbm.at[1, wid], dsti, semi)
    cpo = pltpu.async_copy(ones_hbm, onesv, semi)

    @pl.when(s < NWCH)
    def _zero():
        pltpu.sync_copy(zrows_hbm, acc.at[pl.ds(s * WCH, WCH)])

    cpd.wait()
    cpo.wait()
    plsc.subcore_barrier()
    scats = []
    for blk in range(NBLK):
        scats.append(pltpu.async_copy(onesv, acc.at[dsti.at[blk]], semi,
                                      add=True))
    for cp in scats:
        cp.wait()
    plsc.subcore_barrier()

    @pl.when(s < NWCH)
    def _writeout():
        pltpu.sync_copy(acc.at[pl.ds(s * WCH, WCH)],
                        out_hbm.at[pl.ds(c * N + s * WCH, WCH)])


@functools.partial(
    pl.kernel,
    out_type=jax.ShapeDtypeStruct((NC * N, HID), jnp.float32),
    mesh=_MESH,
    scratch_types=[
        pltpu.VMEM_SHARED((N, HID), jnp.float32),
        pltpu.VMEM((NBLK, CHUNK), jnp.int32),
        pltpu.VMEM((NBLK, CHUNK), jnp.int32),
        pltpu.VMEM((2, CHUNK, HID), jnp.float32),
        pltpu.SemaphoreType.DMA,
        pltpu.SemaphoreType.DMA,
        pltpu.SemaphoreType.DMA,
        pltpu.SemaphoreType.DMA,
        pltpu.SemaphoreType.DMA,
    ],
    compiler_params=pltpu.CompilerParams(use_tc_tiling_on_sc=False),
)
def _edge_aggregate(table_hbm, ei_hbm, zrows_hbm, out_hbm,
                    acc, srci, dsti, rowsv, semi, semg0, semg1,
                    sems0, sems1):
    c = lax.axis_index("c")
    s = lax.axis_index("s")
    wid = c * NS + s
    semg = (semg0, semg1)
    sems = (sems0, sems1)
    NBUF = 2

    cpi = pltpu.async_copy(ei_hbm.at[0, wid], srci, semi)
    cpd = pltpu.async_copy(ei_hbm.at[1, wid], dsti, semi)

    @pl.when(s < NWCH)
    def _zero():
        pltpu.sync_copy(zrows_hbm, acc.at[pl.ds(s * WCH, WCH)])

    cpi.wait()
    cpd.wait()
    gathers = [None] * NBUF
    scats = [None] * NBUF
    gathers[0] = pltpu.async_copy(table_hbm.at[srci.at[0]], rowsv.at[0],
                                  semg[0])
    plsc.subcore_barrier()
    for blk in range(NBLK):
        p = blk % NBUF
        gathers[p].wait()
        scats[p] = pltpu.async_copy(rowsv.at[p], acc.at[dsti.at[blk]],
                                    sems[p], add=True)
        if blk + 1 < NBLK:
            q = (blk + 1) % NBUF
            if blk >= 1:
                scats[q].wait()
            gathers[q] = pltpu.async_copy(table_hbm.at[srci.at[blk + 1]],
                                          rowsv.at[q], semg[q])
    for blk in range(NBLK - NBUF, NBLK):
        scats[blk % NBUF].wait()
    plsc.subcore_barrier()

    @pl.when(s < NWCH)
    def _writeout():
        pltpu.sync_copy(acc.at[pl.ds(s * WCH, WCH)],
                        out_hbm.at[pl.ds(c * N + s * WCH, WCH)])


def _xw1_body(x4_ref, w14_ref, xw_ref):
    xw_ref[...] = jnp.dot(x4_ref[...], w14_ref[...],
                          preferred_element_type=jnp.float32)


def _xw1(x4, w14):
    return pl.pallas_call(
        _xw1_body,
        out_shape=jax.ShapeDtypeStruct((R4, PW), jnp.float32),
    )(x4, w14)


def _prep_body(degp_ref, xw_ref, dinv_ref, xws_ref):
    deg = degp_ref[0] + degp_ref[1] + 1.0
    dinv = lax.rsqrt(deg)
    dinv_ref[...] = dinv
    xws_ref[...] = xw_ref[...] * dinv


def _prep(degp, xw):
    return pl.pallas_call(
        _prep_body,
        out_shape=[
            jax.ShapeDtypeStruct((R4, PW), jnp.float32),
            jax.ShapeDtypeStruct((R4, PW), jnp.float32),
        ],
    )(degp, xw)


def _boundary_body(has_jk, p_ref, xws_ref, dinv_ref, b_ref, w_ref, jk_ref,
                   xwsn_ref, jko_ref):
    total = p_ref[0] + p_ref[1] + xws_ref[...]
    h = jnp.maximum(total * dinv_ref[...] + b_ref[...], 0.0)
    jko = jnp.maximum(jk_ref[...], h) if has_jk else h
    jko_ref[...] = jko
    xwsn_ref[...] = jnp.dot(h, w_ref[...],
                            preferred_element_type=jnp.float32) * dinv_ref[...]


def _boundary(partials, xws, dinv, b, w_next, jk):
    has_jk = jk is not None
    args = [partials, xws, dinv, b, w_next] + ([jk] if has_jk else [])
    if has_jk:
        body = functools.partial(_boundary_body, True)
    else:
        def body(p, xw, di, bb, ww, xn, jo):
            _boundary_body(False, p, xw, di, bb, ww, None, xn, jo)
    return pl.pallas_call(
        body,
        out_shape=[
            jax.ShapeDtypeStruct((R4, PW), jnp.float32),
            jax.ShapeDtypeStruct((R4, PW), jnp.float32),
        ],
    )(*args)


def _final_body(p_ref, xws_ref, dinv_ref, b_ref, jk_ref, fcw_ref,
                fcb_ref, bones_ref, out_ref):
    total = p_ref[0] + p_ref[1] + xws_ref[...]
    h = jnp.maximum(total * dinv_ref[...] + b_ref[...], 0.0)
    jk = jnp.maximum(jk_ref[...], h)
    logits = jnp.dot(jk, fcw_ref[...],
                     preferred_element_type=jnp.float32) + fcb_ref[...]
    m = jnp.max(logits, axis=1, keepdims=True)
    z = logits - m
    ssum = jnp.dot(jnp.exp(z), bones_ref[...],
                   preferred_element_type=jnp.float32)
    out_ref[...] = z - jnp.log(ssum)


def _final(partials, xws, dinv, b, jk, fcw4, fcb4, bones):
    nc4 = fcw4.shape[1]
    return pl.pallas_call(
        _final_body,
        out_shape=jax.ShapeDtypeStruct((R4, nc4), jnp.float32),
    )(partials, xws, dinv, b, jk, fcw4, fcb4, bones)


def _blkdiag(w):
    a, b = w.shape
    out = jnp.zeros((PACK * a, PACK * b), w.dtype)
    for j in range(PACK):
        out = out.at[j * a:(j + 1) * a, j * b:(j + 1) * b].set(w)
    return out


def kernel(x, edge_index, Ws, bs, fcW, fcb):
    ei4 = edge_index.reshape(2, NW, NBLK, CHUNK)
    ones_r = jnp.ones((CHUNK, HID), jnp.float32)
    zeros_r = jnp.zeros((WCH, HID), jnp.float32)

    w14 = _blkdiag(Ws[0])
    w4s = [_blkdiag(w) for w in Ws[1:]]
    fcw4 = _blkdiag(fcW)
    b4s = [jnp.tile(b, PACK).reshape(1, PW) for b in bs]
    fcb4 = jnp.tile(fcb, PACK).reshape(1, -1)

    nclass = fcW.shape[1]
    bones = _blkdiag(jnp.ones((nclass, nclass), jnp.float32))

    deg4 = _degree_hist(ei4, ones_r, zeros_r).reshape(NC, R4, PW)
    x4 = x.reshape(R4, PACK * x.shape[1])
    xw1 = _xw1(x4, w14)
    dinv, xws = _prep(deg4, xw1)

    jk = None
    nlayers = len(Ws)
    for l in range(nlayers):
        partials = _edge_aggregate(xws.reshape(N, HID), ei4, zeros_r)
        p4 = partials.reshape(NC, R4, PW)
        if l + 1 < nlayers:
            xws, jk = _boundary(p4, xws, dinv, b4s[l], w4s[l], jk)
        else:
            out4 = _final(p4, xws, dinv, b4s[l], jk, fcw4, fcb4, bones)
    return out4.reshape(N, nclass)


# speedup vs baseline: 1.0981x; 1.0043x over previous
import functools

import jax
import jax.numpy as jnp
from jax import lax
from jax.experimental import pallas as pl
from jax.experimental.pallas import tpu as pltpu
from jax.experimental.pallas import tpu_sc as plsc

N = 10000
E = 320000
HID = 32
PACK = 4
R4 = N // PACK
PW = PACK * HID
NC = 2
NS = 16
NW = NC * NS
EPT = E // NW
CHUNK = 1000
NBLK = EPT // CHUNK
WCH = 1000
NWCH = N // WCH

_MESH = plsc.VectorSubcoreMesh(core_axis_name="c", subcore_axis_name="s")


@functools.partial(
    pl.kernel,
    out_type=jax.ShapeDtypeStruct((NC * N, HID), jnp.float32),
    mesh=_MESH,
    scratch_types=[
        pltpu.VMEM_SHARED((N, HID), jnp.float32),
        pltpu.VMEM((NBLK, CHUNK), jnp.int32),
        pltpu.VMEM((CHUNK, HID), jnp.float32),
        pltpu.SemaphoreType.DMA,
    ],
    compiler_params=pltpu.CompilerParams(use_tc_tiling_on_sc=False),
)
def _degree_hist(ei_hbm, ones_hbm, zrows_hbm, out_hbm, acc, dsti, onesv,
                 semi):
    c = lax.axis_index("c")
    s = lax.axis_index("s")
    wid = c * NS + s
    cpd = pltpu.async_copy(ei_hbm.at[1, wid], dsti, semi)
    cpo = pltpu.async_copy(ones_hbm, onesv, semi)

    @pl.when(s < NWCH)
    def _zero():
        pltpu.sync_copy(zrows_hbm, acc.at[pl.ds(s * WCH, WCH)])

    cpd.wait()
    cpo.wait()
    plsc.subcore_barrier()
    scats = []
    for blk in range(NBLK):
        scats.append(pltpu.async_copy(onesv, acc.at[dsti.at[blk]], semi,
                                      add=True))
    for cp in scats:
        cp.wait()
    plsc.subcore_barrier()

    @pl.when(s < NWCH)
    def _writeout():
        pltpu.sync_copy(acc.at[pl.ds(s * WCH, WCH)],
                        out_hbm.at[pl.ds(c * N + s * WCH, WCH)])


@functools.partial(
    pl.kernel,
    out_type=jax.ShapeDtypeStruct((NC * N, HID), jnp.float32),
    mesh=_MESH,
    scratch_types=[
        pltpu.VMEM_SHARED((N, HID), jnp.float32),
        pltpu.VMEM((NBLK, CHUNK), jnp.int32),
        pltpu.VMEM((NBLK, CHUNK), jnp.int32),
        pltpu.VMEM((2, CHUNK, HID), jnp.float32),
        pltpu.SemaphoreType.DMA,
        pltpu.SemaphoreType.DMA,
        pltpu.SemaphoreType.DMA,
        pltpu.SemaphoreType.DMA,
        pltpu.SemaphoreType.DMA,
    ],
    compiler_params=pltpu.CompilerParams(use_tc_tiling_on_sc=False),
)
def _edge_aggregate(table_hbm, ei_hbm, zrows_hbm, out_hbm,
                    acc, srci, dsti, rowsv, semi, semg0, semg1,
                    sems0, sems1):
    c = lax.axis_index("c")
    s = lax.axis_index("s")
    wid = c * NS + s
    semg = (semg0, semg1)
    sems = (sems0, sems1)
    NBUF = 2

    cpi = pltpu.async_copy(ei_hbm.at[0, wid], srci, semi)
    cpd = pltpu.async_copy(ei_hbm.at[1, wid], dsti, semi)

    @pl.when(s < NWCH)
    def _zero():
        pltpu.sync_copy(zrows_hbm, acc.at[pl.ds(s * WCH, WCH)])

    cpi.wait()
    cpd.wait()
    gathers = [None] * NBUF
    scats = [None] * NBUF
    gathers[0] = pltpu.async_copy(table_hbm.at[srci.at[0]], rowsv.at[0],
                                  semg[0])
    plsc.subcore_barrier()
    for blk in range(NBLK):
        p = blk % NBUF
        gathers[p].wait()
        scats[p] = pltpu.async_copy(rowsv.at[p], acc.at[dsti.at[blk]],
                                    sems[p], add=True)
        if blk + 1 < NBLK:
            q = (blk + 1) % NBUF
            if blk >= 1:
                scats[q].wait()
            gathers[q] = pltpu.async_copy(table_hbm.at[srci.at[blk + 1]],
                                          rowsv.at[q], semg[q])
    for blk in range(NBLK - NBUF, NBLK):
        scats[blk % NBUF].wait()
    plsc.subcore_barrier()

    @pl.when(s < NWCH)
    def _writeout():
        pltpu.sync_copy(acc.at[pl.ds(s * WCH, WCH)],
                        out_hbm.at[pl.ds(c * N + s * WCH, WCH)])


def _xw1_body(x4_ref, w14_ref, xw_ref):
    xw_ref[...] = jnp.dot(x4_ref[...], w14_ref[...],
                          preferred_element_type=jnp.float32)


def _xw1(x4, w14):
    return pl.pallas_call(
        _xw1_body,
        out_shape=jax.ShapeDtypeStruct((R4, PW), jnp.float32),
    )(x4, w14)


def _prep_body(degp_ref, xw_ref, dinv_ref, xws_ref):
    deg = degp_ref[0] + degp_ref[1] + 1.0
    dinv = lax.rsqrt(deg)
    dinv_ref[...] = dinv
    xws_ref[...] = xw_ref[...] * dinv


def _prep(degp, xw):
    return pl.pallas_call(
        _prep_body,
        out_shape=[
            jax.ShapeDtypeStruct((R4, PW), jnp.float32),
            jax.ShapeDtypeStruct((R4, PW), jnp.float32),
        ],
    )(degp, xw)


def _boundary_body(has_jk, p_ref, xws_ref, dinv_ref, b_ref, w_ref, jk_ref,
                   xwsn_ref, jko_ref):
    total = p_ref[0] + p_ref[1] + xws_ref[...]
    h = jnp.maximum(total * dinv_ref[...] + b_ref[...], 0.0)
    jko = jnp.maximum(jk_ref[...], h) if has_jk else h
    jko_ref[...] = jko
    xwsn_ref[...] = jnp.dot(h, w_ref[...],
                            preferred_element_type=jnp.float32) * dinv_ref[...]


def _boundary(partials, xws, dinv, b, w_next, jk):
    has_jk = jk is not None
    args = [partials, xws, dinv, b, w_next] + ([jk] if has_jk else [])
    if has_jk:
        body = functools.partial(_boundary_body, True)
    else:
        def body(p, xw, di, bb, ww, xn, jo):
            _boundary_body(False, p, xw, di, bb, ww, None, xn, jo)
    return pl.pallas_call(
        body,
        out_shape=[
            jax.ShapeDtypeStruct((R4, PW), jnp.float32),
            jax.ShapeDtypeStruct((R4, PW), jnp.float32),
        ],
    )(*args)


def _final_body(p_ref, xws_ref, dinv_ref, b_ref, jk_ref, fcw_ref,
                fcb_ref, bones_ref, out_ref):
    total = p_ref[0] + p_ref[1] + xws_ref[...]
    h = jnp.maximum(total * dinv_ref[...] + b_ref[...], 0.0)
    jk = jnp.maximum(jk_ref[...], h)
    logits = jnp.dot(jk, fcw_ref[...],
                     preferred_element_type=jnp.float32) + fcb_ref[...]
    m = jnp.max(logits, axis=1, keepdims=True)
    z = logits - m
    ssum = jnp.dot(jnp.exp(z), bones_ref[...],
                   preferred_element_type=jnp.float32)
    out_ref[...] = z - jnp.log(ssum)


def _final(partials, xws, dinv, b, jk, fcw4, fcb4, bones):
    nc4 = fcw4.shape[1]
    return pl.pallas_call(
        _final_body,
        out_shape=jax.ShapeDtypeStruct((R4, nc4), jnp.float32),
    )(partials, xws, dinv, b, jk, fcw4, fcb4, bones)


def _blkdiag(w):
    return jnp.kron(jnp.eye(PACK, dtype=w.dtype), w)


def kernel(x, edge_index, Ws, bs, fcW, fcb):
    ei4 = edge_index.reshape(2, NW, NBLK, CHUNK)
    ones_r = jnp.ones((CHUNK, HID), jnp.float32)
    zeros_r = jnp.zeros((WCH, HID), jnp.float32)

    w14 = _blkdiag(Ws[0])
    w4s = [_blkdiag(w) for w in Ws[1:]]
    fcw4 = _blkdiag(fcW)
    b4s = [jnp.tile(b, PACK).reshape(1, PW) for b in bs]
    fcb4 = jnp.tile(fcb, PACK).reshape(1, -1)

    nclass = fcW.shape[1]
    bones = _blkdiag(jnp.ones((nclass, nclass), jnp.float32))

    deg4 = _degree_hist(ei4, ones_r, zeros_r).reshape(NC, R4, PW)
    x4 = x.reshape(R4, PACK * x.shape[1])
    xw1 = _xw1(x4, w14)
    dinv, xws = _prep(deg4, xw1)

    jk = None
    nlayers = len(Ws)
    for l in range(nlayers):
        partials = _edge_aggregate(xws.reshape(N, HID), ei4, zeros_r)
        p4 = partials.reshape(NC, R4, PW)
        if l + 1 < nlayers:
            xws, jk = _boundary(p4, xws, dinv, b4s[l], w4s[l], jk)
        else:
            out4 = _final(p4, xws, dinv, b4s[l], jk, fcw4, fcb4, bones)
    return out4.reshape(N, nclass)
